# baseline (speedup 1.0000x reference)
import functools

import jax
import jax.numpy as jnp
from jax import lax
from jax.experimental import pallas as pl
from jax.experimental.pallas import tpu as pltpu
from jax.experimental.pallas import tpu_sc as plsc

N = 320000
D = 128
NSEG = 4096
NC = 2
NS = 16
NW = NC * NS
PPW = N // NW
B = 128
NB = PPW // B
TAIL = PPW - NB * B
RPT = NSEG // NS
NBUF = 4
AHEAD = NBUF - 2

_mesh = plsc.VectorSubcoreMesh(core_axis_name="c", subcore_axis_name="s")


@functools.partial(
    pl.kernel,
    out_type=jax.ShapeDtypeStruct((NC * NSEG, D), jnp.float32),
    mesh=_mesh,
    scratch_types=[
        pltpu.VMEM((PPW,), jnp.int32),
        pltpu.VMEM((PPW,), jnp.int32),
        pltpu.VMEM((PPW,), jnp.int32),
        [pltpu.VMEM((B,), jnp.int32) for _ in range(NBUF)],
        pltpu.VMEM((TAIL,), jnp.int32),
        [pltpu.VMEM((B, D), jnp.float32) for _ in range(NBUF)],
        pltpu.VMEM_SHARED((NSEG, D), jnp.float32),
        [pltpu.SemaphoreType.DMA for _ in range(NBUF)],
        [pltpu.SemaphoreType.DMA for _ in range(NBUF)],
    ],
)
def _chunk_sum_sc(values_hbm, coords_hbm, out_hbm, cx_v, cy_v, cz_v, idx_v,
                  idxt_v, rows_v, acc, gsem, ssem):
    c = lax.axis_index("c")
    s = lax.axis_index("s")
    wid = s * NC + c
    base0 = wid * PPW

    pltpu.async_copy(coords_hbm.at[pl.ds(base0, PPW)], cx_v, gsem[0])
    pltpu.async_copy(coords_hbm.at[pl.ds(N + base0, PPW)], cy_v, gsem[1])
    pltpu.async_copy(coords_hbm.at[pl.ds(2 * N + base0, PPW)], cz_v, gsem[2])

    zero16 = jnp.zeros((16,), jnp.float32)

    def _zero_body(i, _):
        rows_v[0][i // (D // 16), pl.ds((i % (D // 16)) * 16, 16)] = zero16
        return 0

    lax.fori_loop(0, B * (D // 16), _zero_body, 0)
    pltpu.sync_copy(rows_v[0], acc.at[pl.ds(s * RPT, B)])
    pltpu.sync_copy(rows_v[0], acc.at[pl.ds(s * RPT + B, B)])
    pltpu.make_async_copy(coords_hbm.at[pl.ds(0, PPW)], cx_v, gsem[0]).wait()
    pltpu.make_async_copy(coords_hbm.at[pl.ds(0, PPW)], cy_v, gsem[1]).wait()
    pltpu.make_async_copy(coords_hbm.at[pl.ds(0, PPW)], cz_v, gsem[2]).wait()
    plsc.subcore_barrier()

    def _compute_ids(k, p):
        off = k * B
        for g in range(B // 16):
            c0 = cx_v[pl.ds(off + g * 16, 16)]
            c1 = cy_v[pl.ds(off + g * 16, 16)]
            c2 = cz_v[pl.ds(off + g * 16, 16)]
            idx_v[p][pl.ds(g * 16, 16)] = ((c0 >> 4) << 8) | ((c1 >> 4) << 4) | (c2 >> 4)

    def _issue_gather(k, p):
        pltpu.async_copy(values_hbm.at[pl.ds(base0 + k * B, B)], rows_v[p],
                         gsem[p])

    def _wait_gather(p):
        pltpu.make_async_copy(values_hbm.at[pl.ds(0, B)], rows_v[p],
                              gsem[p]).wait()

    def _issue_scatter(p):
        pltpu.async_copy(rows_v[p], acc.at[idx_v[p]], ssem[p], add=True)

    def _wait_scatter(p):
        pltpu.make_async_copy(rows_v[p], acc.at[idx_v[p]], ssem[p]).wait()

    def _step(k, p, first):
        r = (p + AHEAD) % NBUF
        _wait_gather(p)
        _compute_ids(k, p)
        _issue_scatter(p)
        if not first:
            _wait_scatter(r)

        @pl.when(k + AHEAD < NB)
        def _():
            _issue_gather(k + AHEAD, r)

    for k in range(AHEAD):
        _issue_gather(k, k)
    for k in range(AHEAD):
        _step(k, k, k < 2)

    def _outer(ko, _):
        for b in range(NBUF):
            k = NBUF * ko + AHEAD + b
            _step(k, (AHEAD + b) % NBUF, False)
        return 0

    lax.fori_loop(0, (NB - AHEAD) // NBUF, _outer, 0)
    _wait_scatter((NB - 2) % NBUF)
    _wait_scatter((NB - 1) % NBUF)

    baset = base0 + NB * B
    offt = NB * B
    c0 = cx_v[pl.ds(offt, TAIL)]
    c1 = cy_v[pl.ds(offt, TAIL)]
    c2 = cz_v[pl.ds(offt, TAIL)]
    idxt_v[...] = ((c0 >> 4) << 8) | ((c1 >> 4) << 4) | (c2 >> 4)
    pltpu.sync_copy(values_hbm.at[pl.ds(baset, TAIL)],
                    rows_v[0].at[pl.ds(0, TAIL)])
    pltpu.sync_copy(rows_v[0].at[pl.ds(0, TAIL)], acc.at[idxt_v], add=True)

    plsc.subcore_barrier()
    pltpu.sync_copy(acc.at[pl.ds(s * RPT, RPT)],
                    out_hbm.at[pl.ds(c * NSEG + s * RPT, RPT)])


def _add_partials(p_ref, o_ref):
    o_ref[...] = p_ref[0] + p_ref[1]


def kernel(values, coords):
    coords_t = coords.T.reshape(-1)
    partial = _chunk_sum_sc(values, coords_t)
    return pl.pallas_call(
        _add_partials,
        out_shape=jax.ShapeDtypeStruct((NSEG, D), jnp.float32),
    )(partial.reshape(NC, NSEG, D))


# speedup vs baseline: 1.0366x; 1.0366x over previous
import functools

import jax
import jax.numpy as jnp
from jax import lax
from jax.experimental import pallas as pl
from jax.experimental.pallas import tpu as pltpu
from jax.experimental.pallas import tpu_sc as plsc

N = 320000
D = 128
NSEG = 4096
NC = 2
NS = 16
NW = NC * NS
PPW = N // NW
B = 128
NB = PPW // B
TAIL = PPW - NB * B
RPT = NSEG // NS
NBUF = 5
AHEAD = NBUF - 2
CP = 400
NCH = PPW // CP

_mesh = plsc.VectorSubcoreMesh(core_axis_name="c", subcore_axis_name="s")


@functools.partial(
    pl.kernel,
    out_type=jax.ShapeDtypeStruct((NC * NSEG, D), jnp.float32),
    mesh=_mesh,
    scratch_types=[
        pltpu.VMEM((PPW,), jnp.int32),
        [pltpu.VMEM((CP,), jnp.int32) for _ in range(6)],
        [pltpu.VMEM((B, D), jnp.float32) for _ in range(NBUF)],
        pltpu.VMEM_SHARED((NSEG, D), jnp.float32),
        [pltpu.SemaphoreType.DMA for _ in range(NBUF)],
        [pltpu.SemaphoreType.DMA for _ in range(NBUF)],
        [pltpu.SemaphoreType.DMA for _ in range(6)],
    ],
)
def _chunk_sum_sc(values_hbm, coords_hbm, out_hbm, ids_v, cb, rows_v, acc,
                  gsem, ssem, csem):
    c = lax.axis_index("c")
    s = lax.axis_index("s")
    wid = s * NC + c
    base0 = wid * PPW

    def _issue_chunk(j, q):
        off = base0 + j * CP
        pltpu.async_copy(coords_hbm.at[pl.ds(off, CP)], cb[3 * q], csem[3 * q])
        pltpu.async_copy(coords_hbm.at[pl.ds(N + off, CP)], cb[3 * q + 1],
                         csem[3 * q + 1])
        pltpu.async_copy(coords_hbm.at[pl.ds(2 * N + off, CP)], cb[3 * q + 2],
                         csem[3 * q + 2])

    def _wait_chunk(q):
        for a in range(3):
            pltpu.make_async_copy(coords_hbm.at[pl.ds(0, CP)], cb[3 * q + a],
                                  csem[3 * q + a]).wait()

    def _ids_chunk(j, q):
        for g in range(CP // 16):
            c0 = cb[3 * q][pl.ds(g * 16, 16)]
            c1 = cb[3 * q + 1][pl.ds(g * 16, 16)]
            c2 = cb[3 * q + 2][pl.ds(g * 16, 16)]
            ids_v[pl.ds(j * CP + g * 16, 16)] = (
                ((c0 >> 4) << 8) | ((c1 >> 4) << 4) | (c2 >> 4))

    _issue_chunk(0, 0)
    _issue_chunk(1, 1)

    zero16 = jnp.zeros((16,), jnp.float32)

    def _zero_body(i, _):
        rows_v[0][i // (D // 16), pl.ds((i % (D // 16)) * 16, 16)] = zero16
        return 0

    lax.fori_loop(0, B * (D // 16), _zero_body, 0)
    pltpu.sync_copy(rows_v[0], acc.at[pl.ds(s * RPT, B)])
    pltpu.sync_copy(rows_v[0], acc.at[pl.ds(s * RPT + B, B)])

    def _chunk_pair(jo, _):
        j0 = 2 * jo
        _wait_chunk(0)
        _ids_chunk(j0, 0)

        @pl.when(j0 + 2 < NCH)
        def _():
            _issue_chunk(j0 + 2, 0)

        _wait_chunk(1)
        _ids_chunk(j0 + 1, 1)

        @pl.when(j0 + 3 < NCH)
        def _():
            _issue_chunk(j0 + 3, 1)

        return 0

    lax.fori_loop(0, (NCH - 1) // 2, _chunk_pair, 0)
    _wait_chunk(0)
    _ids_chunk(NCH - 1, 0)
    plsc.subcore_barrier()

    def _issue_gather(k, p):
        pltpu.async_copy(values_hbm.at[pl.ds(base0 + k * B, B)], rows_v[p],
                         gsem[p])

    def _wait_gather(p):
        pltpu.make_async_copy(values_hbm.at[pl.ds(0, B)], rows_v[p],
                              gsem[p]).wait()

    def _issue_scatter(k, p):
        pltpu.async_copy(rows_v[p], acc.at[ids_v.at[pl.ds(k * B, B)]],
                         ssem[p], add=True)

    def _wait_scatter(k, p):
        pltpu.make_async_copy(rows_v[p], acc.at[ids_v.at[pl.ds(k * B, B)]],
                              ssem[p]).wait()

    def _step(k, p, first):
        r = (p + AHEAD) % NBUF
        _wait_gather(p)
        _issue_scatter(k, p)
        if not first:
            _wait_scatter(k - 2, r)

        @pl.when(k + AHEAD < NB)
        def _():
            _issue_gather(k + AHEAD, r)

    for k in range(AHEAD):
        _issue_gather(k, k)
    for k in range(AHEAD):
        _step(k, k, k < 2)

    def _outer(ko, _):
        for b in range(NBUF):
            k = NBUF * ko + AHEAD + b
            _step(k, (AHEAD + b) % NBUF, False)
        return 0

    lax.fori_loop(0, (NB - AHEAD) // NBUF, _outer, 0)
    _wait_scatter(NB - 2, (NB - 2) % NBUF)
    _wait_scatter(NB - 1, (NB - 1) % NBUF)

    baset = base0 + NB * B
    pltpu.sync_copy(values_hbm.at[pl.ds(baset, TAIL)],
                    rows_v[0].at[pl.ds(0, TAIL)])
    pltpu.sync_copy(rows_v[0].at[pl.ds(0, TAIL)],
                    acc.at[ids_v.at[pl.ds(NB * B, TAIL)]], add=True)

    plsc.subcore_barrier()
    pltpu.sync_copy(acc.at[pl.ds(s * RPT, RPT)],
                    out_hbm.at[pl.ds(c * NSEG + s * RPT, RPT)])


def _add_partials(p_ref, o_ref):
    o_ref[...] = p_ref[0] + p_ref[1]


def kernel(values, coords):
    coords_t = coords.T.reshape(-1)
    partial = _chunk_sum_sc(values, coords_t)
    return pl.pallas_call(
        _add_partials,
        out_shape=jax.ShapeDtypeStruct((NSEG, D), jnp.float32),
    )(partial.reshape(NC, NSEG, D))


# speedup vs baseline: 1.0531x; 1.0158x over previous
import functools

import jax
import jax.numpy as jnp
from jax import lax
from jax.experimental import pallas as pl
from jax.experimental.pallas import tpu as pltpu
from jax.experimental.pallas import tpu_sc as plsc

N = 320000
D = 128
NSEG = 4096
NC = 2
NS = 16
NW = NC * NS
PPW = N // NW
B = 128
NB = PPW // B
TAIL = PPW - NB * B
RPT = NSEG // NS
NBUF = 5
AHEAD = NBUF - 2
CP = 400
NCH = PPW // CP

_mesh = plsc.VectorSubcoreMesh(core_axis_name="c", subcore_axis_name="s")


@functools.partial(
    pl.kernel,
    out_type=jax.ShapeDtypeStruct((NC * NSEG, D), jnp.float32),
    mesh=_mesh,
    scratch_types=[
        pltpu.VMEM((PPW,), jnp.int32),
        [pltpu.VMEM((CP,), jnp.int32) for _ in range(6)],
        [pltpu.VMEM((B, D), jnp.float32) for _ in range(NBUF)],
        pltpu.VMEM_SHARED((NSEG, D), jnp.float32),
        [pltpu.SemaphoreType.DMA for _ in range(NBUF)],
        [pltpu.SemaphoreType.DMA for _ in range(NBUF)],
        [pltpu.SemaphoreType.DMA for _ in range(6)],
    ],
)
def _chunk_sum_sc(values_hbm, coords_hbm, out_hbm, ids_v, cb, rows_v, acc,
                  gsem, ssem, csem):
    c = lax.axis_index("c")
    s = lax.axis_index("s")
    wid = s * NC + c
    base0 = wid * PPW

    def _issue_chunk(j, q):
        off = base0 + j * CP
        pltpu.async_copy(coords_hbm.at[pl.ds(off, CP)], cb[3 * q], csem[3 * q])
        pltpu.async_copy(coords_hbm.at[pl.ds(N + off, CP)], cb[3 * q + 1],
                         csem[3 * q + 1])
        pltpu.async_copy(coords_hbm.at[pl.ds(2 * N + off, CP)], cb[3 * q + 2],
                         csem[3 * q + 2])

    def _wait_chunk(q):
        for a in range(3):
            pltpu.make_async_copy(coords_hbm.at[pl.ds(0, CP)], cb[3 * q + a],
                                  csem[3 * q + a]).wait()

    def _ids_chunk(j, q):
        for g in range(CP // 16):
            c0 = cb[3 * q][pl.ds(g * 16, 16)]
            c1 = cb[3 * q + 1][pl.ds(g * 16, 16)]
            c2 = cb[3 * q + 2][pl.ds(g * 16, 16)]
            ids_v[pl.ds(j * CP + g * 16, 16)] = (
                ((c0 >> 4) << 8) | ((c1 >> 4) << 4) | (c2 >> 4))

    _issue_chunk(0, 0)
    _issue_chunk(1, 1)

    zero16 = jnp.zeros((16,), jnp.float32)

    def _zero_body(i, _):
        rows_v[0][i // (D // 16), pl.ds((i % (D // 16)) * 16, 16)] = zero16
        return 0

    lax.fori_loop(0, B * (D // 16), _zero_body, 0)
    pltpu.sync_copy(rows_v[0], acc.at[pl.ds(s * RPT, B)])
    pltpu.sync_copy(rows_v[0], acc.at[pl.ds(s * RPT + B, B)])

    for k0 in range(AHEAD):
        pltpu.async_copy(values_hbm.at[pl.ds(base0 + k0 * B, B)], rows_v[k0],
                         gsem[k0])

    def _chunk_pair(jo, _):
        j0 = 2 * jo
        _wait_chunk(0)
        _ids_chunk(j0, 0)

        @pl.when(j0 + 2 < NCH)
        def _():
            _issue_chunk(j0 + 2, 0)

        _wait_chunk(1)
        _ids_chunk(j0 + 1, 1)

        @pl.when(j0 + 3 < NCH)
        def _():
            _issue_chunk(j0 + 3, 1)

        return 0

    lax.fori_loop(0, (NCH - 1) // 2, _chunk_pair, 0)
    _wait_chunk(0)
    _ids_chunk(NCH - 1, 0)
    plsc.subcore_barrier()

    def _issue_gather(k, p):
        pltpu.async_copy(values_hbm.at[pl.ds(base0 + k * B, B)], rows_v[p],
                         gsem[p])

    def _wait_gather(p):
        pltpu.make_async_copy(values_hbm.at[pl.ds(0, B)], rows_v[p],
                              gsem[p]).wait()

    def _issue_scatter(k, p):
        pltpu.async_copy(rows_v[p], acc.at[ids_v.at[pl.ds(k * B, B)]],
                         ssem[p], add=True)

    def _wait_scatter(k, p):
        pltpu.make_async_copy(rows_v[p], acc.at[ids_v.at[pl.ds(k * B, B)]],
                              ssem[p]).wait()

    def _step(k, p, first):
        r = (p + AHEAD) % NBUF
        _wait_gather(p)
        _issue_scatter(k, p)
        if not first:
            _wait_scatter(k - 2, r)

        @pl.when(k + AHEAD < NB)
        def _():
            _issue_gather(k + AHEAD, r)

    for k in range(AHEAD):
        _step(k, k, k < 2)

    def _outer(ko, _):
        for b in range(NBUF):
            k = NBUF * ko + AHEAD + b
            _step(k, (AHEAD + b) % NBUF, False)
        return 0

    lax.fori_loop(0, (NB - AHEAD) // NBUF, _outer, 0)
    _wait_scatter(NB - 2, (NB - 2) % NBUF)
    _wait_scatter(NB - 1, (NB - 1) % NBUF)

    baset = base0 + NB * B
    pltpu.sync_copy(values_hbm.at[pl.ds(baset, TAIL)],
                    rows_v[0].at[pl.ds(0, TAIL)])
    pltpu.sync_copy(rows_v[0].at[pl.ds(0, TAIL)],
                    acc.at[ids_v.at[pl.ds(NB * B, TAIL)]], add=True)

    plsc.subcore_barrier()
    pltpu.sync_copy(acc.at[pl.ds(s * RPT, RPT)],
                    out_hbm.at[pl.ds(c * NSEG + s * RPT, RPT)])


def _add_partials(p_ref, o_ref):
    o_ref[...] = p_ref[0] + p_ref[1]


def kernel(values, coords):
    coords_t = coords.T.reshape(-1)
    partial = _chunk_sum_sc(values, coords_t)
    return pl.pallas_call(
        _add_partials,
        out_shape=jax.ShapeDtypeStruct((NSEG, D), jnp.float32),
    )(partial.reshape(NC, NSEG, D))
